# baseline (speedup 1.0000x reference)
import jax
import jax.numpy as jnp
from jax.experimental import pallas as pl
from jax.experimental.pallas import tpu as pltpu
from jax.experimental.pallas import tpu_sc as plsc

_N_EXPERTS = 64
_TOP_K = 8


def _pack_kernel(x_ref, wt_ref, key_ref):
    x = x_ref[...]
    wt = wt_ref[...]
    scores = jnp.dot(x, wt, preferred_element_type=jnp.float32)
    b = jax.lax.bitcast_convert_type(scores, jnp.int32)
    mono = b ^ jax.lax.shift_right_logical(
        jax.lax.shift_right_arithmetic(b, 31), 1)
    rev_iota = (_N_EXPERTS - 1) - jax.lax.broadcasted_iota(
        jnp.int32, scores.shape, 1)
    key_ref[...] = (mono & jnp.int32(~(_N_EXPERTS - 1))) | rev_iota


def _tc_keys(x, wt):
    n, h = x.shape
    t = 4096
    return pl.pallas_call(
        _pack_kernel,
        grid=(n // t,),
        in_specs=[
            pl.BlockSpec((t, h), lambda i: (i, 0)),
            pl.BlockSpec((h, _N_EXPERTS), lambda i: (0, 0)),
        ],
        out_specs=pl.BlockSpec((t, _N_EXPERTS), lambda i: (i, 0)),
        out_shape=jax.ShapeDtypeStruct((n, _N_EXPERTS), jnp.int32),
    )(x, wt)


def _merge16(a, b):
    m = jnp.maximum(a, jax.lax.rev(b, (0,)))
    ms, _ = plsc.sort_key_val(m, m, descending=True)
    return ms


def _sc_topk(keys):
    n = keys.shape[0]
    mesh = plsc.VectorSubcoreMesh(core_axis_name="core",
                                  subcore_axis_name="subcore",
                                  num_cores=2, num_subcores=16)
    bt = 64
    kmask = _N_EXPERTS - 1

    @pl.kernel(out_type=[jax.ShapeDtypeStruct((n, 16), jnp.int32),
                         jax.ShapeDtypeStruct((n, 16), jnp.float32)],
               mesh=mesh, scratch_types=[],
               compiler_params=pltpu.CompilerParams(
                   needs_layout_passes=False))
    def topk_kernel(keys_hbm, idx_hbm, w_hbm):
        def body(k_vmem, idx_vmem, w_vmem):
            lane = jax.lax.iota(jnp.int32, 16)

            @pl.loop(0, bt)
            def _(r):
                g = []
                for j in range(4):
                    kj = k_vmem[r, pl.ds(j * 16, 16)]
                    ks, _ = plsc.sort_key_val(kj, kj, descending=True)
                    g.append(ks)
                m = _merge16(_merge16(g[0], g[1]), _merge16(g[2], g[3]))
                ids = (_N_EXPERTS - 1) - (m & kmask)
                vb = m & ~kmask
                vb = vb ^ jax.lax.shift_right_logical(
                    jax.lax.shift_right_arithmetic(vb, 31), 1)
                v = jax.lax.bitcast_convert_type(vb, jnp.float32)
                e = jnp.exp(v - jnp.max(v))
                e = jnp.where(lane < _TOP_K, e, 0.0)
                w = e / jnp.sum(e)
                idx_vmem[r, :] = ids
                w_vmem[r, :] = w

        pltpu.emit_pipeline(
            body,
            grid=(n // bt,),
            in_specs=[pl.BlockSpec(block_shape=(bt, _N_EXPERTS),
                                   index_map=lambda i: (i, 0))],
            out_specs=[pl.BlockSpec(block_shape=(bt, 16),
                                    index_map=lambda i: (i, 0)),
                       pl.BlockSpec(block_shape=(bt, 16),
                                    index_map=lambda i: (i, 0))],
            core_axis_name=("core", "subcore"),
            dimension_semantics=(pltpu.PARALLEL,),
        )(keys_hbm, idx_hbm, w_hbm)

    return topk_kernel(keys)


def kernel(hidden_states, weight):
    x = hidden_states.reshape(-1, hidden_states.shape[-1])
    wt = weight.T
    keys = _tc_keys(x, wt)
    idx16, w16 = _sc_topk(keys)
    return idx16[:, :_TOP_K], w16[:, :_TOP_K]


# speedup vs baseline: 3.6473x; 3.6473x over previous
import jax
import jax.numpy as jnp
from jax.experimental import pallas as pl

_N_EXPERTS = 64
_TOP_K = 8
_SUB = 8


def _sub_gate(x, wt):
    scores = jnp.dot(x, wt, preferred_element_type=jnp.float32)
    st = scores.T
    t = st.shape[1]

    b = jax.lax.bitcast_convert_type(st, jnp.int32)
    mono = b ^ jax.lax.shift_right_logical(
        jax.lax.shift_right_arithmetic(b, 31), 1)
    rev_iota = (_N_EXPERTS - 1) - jax.lax.broadcasted_iota(
        jnp.int32, (_N_EXPERTS, t), 0)
    key = (mono & jnp.int32(~(_N_EXPERTS - 1))) | rev_iota

    picks = []
    for k in range(_TOP_K):
        mk = jnp.max(key, axis=0, keepdims=True)
        picks.append(mk)
        if k < _TOP_K - 1:
            key = jnp.where(key == mk, jnp.int32(-2147483648), key)

    pk = jnp.concatenate(picks, axis=0)
    ids = (_N_EXPERTS - 1) - (pk & jnp.int32(_N_EXPERTS - 1))
    vb = pk & jnp.int32(~(_N_EXPERTS - 1))
    vb = vb ^ jax.lax.shift_right_logical(
        jax.lax.shift_right_arithmetic(vb, 31), 1)
    v = jax.lax.bitcast_convert_type(vb, jnp.float32)
    e = jnp.exp(v - v[0:1, :])
    w = e / jnp.sum(e, axis=0, keepdims=True)
    return ids, w


def _gate_kernel(x_ref, wt_ref, idx_ref, w_ref):
    wt = wt_ref[...]
    t = x_ref.shape[0] // _SUB
    for s in range(_SUB):
        ids, w = _sub_gate(x_ref[s * t:(s + 1) * t, :], wt)
        idx_ref[:, s * t:(s + 1) * t] = ids
        w_ref[:, s * t:(s + 1) * t] = w


def kernel(hidden_states, weight):
    x = hidden_states.reshape(-1, hidden_states.shape[-1])
    n, h = x.shape
    wt = weight.T
    t = 8192
    idx_t, w_t = pl.pallas_call(
        _gate_kernel,
        grid=(n // t,),
        in_specs=[
            pl.BlockSpec((t, h), lambda i: (i, 0)),
            pl.BlockSpec((h, _N_EXPERTS), lambda i: (0, 0)),
        ],
        out_specs=[
            pl.BlockSpec((_TOP_K, t), lambda i: (0, i)),
            pl.BlockSpec((_TOP_K, t), lambda i: (0, i)),
        ],
        out_shape=[
            jax.ShapeDtypeStruct((_TOP_K, n), jnp.int32),
            jax.ShapeDtypeStruct((_TOP_K, n), jnp.float32),
        ],
    )(x, wt)
    return idx_t.T, w_t.T


# speedup vs baseline: 3.9000x; 1.0693x over previous
import jax
import jax.numpy as jnp
from jax.experimental import pallas as pl

_N_EXPERTS = 64
_TOP_K = 8
_SUB = 16


def _sub_gate(x, wt):
    scores = jnp.dot(x, wt, preferred_element_type=jnp.float32)
    st = scores.T
    t = st.shape[1]

    b = jax.lax.bitcast_convert_type(st, jnp.int32)
    mono = b ^ jax.lax.shift_right_logical(
        jax.lax.shift_right_arithmetic(b, 31), 1)
    rev_iota = (_N_EXPERTS - 1) - jax.lax.broadcasted_iota(
        jnp.int32, (_N_EXPERTS, t), 0)
    key = (mono & jnp.int32(~(_N_EXPERTS - 1))) | rev_iota

    picks = []
    for k in range(_TOP_K):
        mk = jnp.max(key, axis=0, keepdims=True)
        picks.append(mk)
        if k < _TOP_K - 1:
            key = jnp.where(key == mk, jnp.int32(-2147483648), key)

    pk = jnp.concatenate(picks, axis=0)
    ids = (_N_EXPERTS - 1) - (pk & jnp.int32(_N_EXPERTS - 1))
    vb = pk & jnp.int32(~(_N_EXPERTS - 1))
    vb = vb ^ jax.lax.shift_right_logical(
        jax.lax.shift_right_arithmetic(vb, 31), 1)
    v = jax.lax.bitcast_convert_type(vb, jnp.float32)
    e = jnp.exp(v - v[0:1, :])
    w = e / jnp.sum(e, axis=0, keepdims=True)
    return ids, w


def _gate_kernel(x_ref, wt_ref, idx_ref, w_ref):
    wt = wt_ref[...]
    t = x_ref.shape[0] // _SUB
    for s in range(_SUB):
        ids, w = _sub_gate(x_ref[s * t:(s + 1) * t, :], wt)
        idx_ref[:, s * t:(s + 1) * t] = ids
        w_ref[:, s * t:(s + 1) * t] = w


def kernel(hidden_states, weight):
    x = hidden_states.reshape(-1, hidden_states.shape[-1])
    n, h = x.shape
    wt = weight.T
    t = 4096
    idx_t, w_t = pl.pallas_call(
        _gate_kernel,
        grid=(n // t,),
        in_specs=[
            pl.BlockSpec((t, h), lambda i: (i, 0)),
            pl.BlockSpec((h, _N_EXPERTS), lambda i: (0, 0)),
        ],
        out_specs=[
            pl.BlockSpec((_TOP_K, t), lambda i: (0, i)),
            pl.BlockSpec((_TOP_K, t), lambda i: (0, i)),
        ],
        out_shape=[
            jax.ShapeDtypeStruct((_TOP_K, n), jnp.int32),
            jax.ShapeDtypeStruct((_TOP_K, n), jnp.float32),
        ],
    )(x, wt)
    return idx_t.T, w_t.T
